# baseline (speedup 1.0000x reference)
import math

import jax
import jax.numpy as jnp
from jax import lax
from jax.experimental import pallas as pl
from jax.experimental.pallas import tpu as pltpu
from jax.experimental.pallas import tpu_sc as plsc

VOCAB = 100000
CTX = 2048
DIM = 768
BATCH = 4
SCALE = math.sqrt(DIM)

ROWS = BATCH * CTX
NW = 32
RPW = ROWS // NW
CHUNK = 32
NCHUNK = RPW // CHUNK
NBUF = 2
LANES = 16
VPR = DIM // LANES


def _emb_body(x_hbm, tab_hbm, pos_hbm, out_hbm, idx_v, rows_v, pos_v,
              gsem0, gsem1, psem0, psem1, ssem0, ssem1):
    gsems = (gsem0, gsem1)
    psems = (psem0, psem1)
    ssems = (ssem0, ssem1)
    cid = lax.axis_index("c")
    sid = lax.axis_index("s")
    wid = sid * 2 + cid
    base = wid * RPW
    pbase = lax.rem(base, CTX)

    pltpu.sync_copy(x_hbm.at[pl.ds(base, RPW)], idx_v)

    def issue(k):
        b = k % NBUF
        g = pltpu.async_copy(
            tab_hbm.at[idx_v.at[pl.ds(k * CHUNK, CHUNK)]],
            rows_v.at[b], gsems[b])
        p = pltpu.async_copy(
            pos_hbm.at[pl.ds(pbase + k * CHUNK, CHUNK)],
            pos_v.at[b], psems[b])
        return g, p

    inflight = [None] * NCHUNK
    stores = [None] * NCHUNK
    inflight[0] = issue(0)
    inflight[1] = issue(1)
    for k in range(NCHUNK):
        b = k % NBUF
        g, p = inflight[k]
        g.wait()
        p.wait()
        buf = rows_v.at[b]
        pbuf = pos_v.at[b]

        @plsc.parallel_loop(0, CHUNK, 1, unroll=2)
        def _row_body(r):
            for j in range(VPR):
                sl = pl.ds(j * LANES, LANES)
                plsc.addupdate(pbuf.at[r, sl], buf[r, sl] * SCALE)
        stores[k] = pltpu.async_copy(
            pbuf, out_hbm.at[pl.ds(base + k * CHUNK, CHUNK)], ssems[b])
        if k + 2 < NCHUNK:
            stores[k].wait()
            inflight[k + 2] = issue(k + 2)
    stores[NCHUNK - 2].wait()
    stores[NCHUNK - 1].wait()


def kernel(x, token_emb, pos_enc):
    x_flat = x.reshape(ROWS).astype(jnp.int32)
    pos2d = pos_enc.reshape(CTX, DIM)

    mesh = plsc.VectorSubcoreMesh(core_axis_name="c", subcore_axis_name="s")
    out = pl.kernel(
        _emb_body,
        mesh=mesh,
        out_type=jax.ShapeDtypeStruct((ROWS, DIM), jnp.float32),
        scratch_types=[
            pltpu.VMEM((RPW,), jnp.int32),
            pltpu.VMEM((NBUF, CHUNK, DIM), jnp.float32),
            pltpu.VMEM((NBUF, CHUNK, DIM), jnp.float32),
            pltpu.SemaphoreType.DMA,
            pltpu.SemaphoreType.DMA,
            pltpu.SemaphoreType.DMA,
            pltpu.SemaphoreType.DMA,
            pltpu.SemaphoreType.DMA,
            pltpu.SemaphoreType.DMA,
        ],
    )(x_flat, token_emb, pos2d)
    return out.reshape(BATCH, CTX, DIM)


# speedup vs baseline: 1.2155x; 1.2155x over previous
import math

import jax
import jax.numpy as jnp
from jax import lax
from jax.experimental import pallas as pl
from jax.experimental.pallas import tpu as pltpu
from jax.experimental.pallas import tpu_sc as plsc

VOCAB = 100000
CTX = 2048
DIM = 768
BATCH = 4
SCALE = math.sqrt(DIM)

ROWS = BATCH * CTX
NW = 32
PPW = CTX // NW
PC = 8
NCHUNK = PPW // PC
NBUF = 2
LANES = 16
VPR = DIM // LANES


def _emb_body(x_hbm, tab_hbm, pos_hbm, out_hbm, idx_v, rows_v, pos_v,
              gsem0, gsem1, ssem0, ssem1):
    gsems = (gsem0, gsem1)
    ssems = (ssem0, ssem1)
    cid = lax.axis_index("c")
    sid = lax.axis_index("s")
    wid = sid * 2 + cid
    pbase = wid * PPW

    pltpu.sync_copy(pos_hbm.at[pl.ds(pbase, PPW)], pos_v)
    for b in range(BATCH):
        pltpu.sync_copy(x_hbm.at[pl.ds(b * CTX + pbase, PPW)], idx_v.at[b])

    def issue(k):
        bsel = k % NBUF
        return [
            pltpu.async_copy(
                tab_hbm.at[idx_v.at[b, pl.ds(k * PC, PC)]],
                rows_v.at[bsel, b], gsems[bsel])
            for b in range(BATCH)
        ]

    inflight = [None] * NCHUNK
    stores = [None] * NCHUNK
    inflight[0] = issue(0)
    inflight[1] = issue(1)
    for k in range(NCHUNK):
        bsel = k % NBUF
        for g in inflight[k]:
            g.wait()
        buf = rows_v.at[bsel]

        @plsc.parallel_loop(0, VPR, 1, unroll=1)
        def _col_body(j):
            sl = pl.ds(j * LANES, LANES)
            for p in range(PC):
                pv = pos_v[k * PC + p, sl]
                for b in range(BATCH):
                    buf[b, p, sl] = buf[b, p, sl] * SCALE + pv

        stores[k] = [
            pltpu.async_copy(
                buf.at[b],
                out_hbm.at[pl.ds(b * CTX + pbase + k * PC, PC)], ssems[bsel])
            for b in range(BATCH)
        ]
        if k + 2 < NCHUNK:
            for s in stores[k]:
                s.wait()
            inflight[k + 2] = issue(k + 2)
    for k in (NCHUNK - 2, NCHUNK - 1):
        for s in stores[k]:
            s.wait()


def kernel(x, token_emb, pos_enc):
    x_flat = x.reshape(ROWS).astype(jnp.int32)
    pos2d = pos_enc.reshape(CTX, DIM)

    mesh = plsc.VectorSubcoreMesh(core_axis_name="c", subcore_axis_name="s")
    out = pl.kernel(
        _emb_body,
        mesh=mesh,
        out_type=jax.ShapeDtypeStruct((ROWS, DIM), jnp.float32),
        scratch_types=[
            pltpu.VMEM((BATCH, PPW), jnp.int32),
            pltpu.VMEM((NBUF, BATCH, PC, DIM), jnp.float32),
            pltpu.VMEM((PPW, DIM), jnp.float32),
            pltpu.SemaphoreType.DMA,
            pltpu.SemaphoreType.DMA,
            pltpu.SemaphoreType.DMA,
            pltpu.SemaphoreType.DMA,
        ],
    )(x_flat, token_emb, pos2d)
    return out.reshape(BATCH, CTX, DIM)
